# baseline (speedup 1.0000x reference)
import functools

import jax
import jax.numpy as jnp
from jax import lax
from jax.experimental import pallas as pl
from jax.experimental.pallas import tpu as pltpu
from jax.experimental.pallas import tpu_sc as plsc

N = 10000
E = 320000
D_IN = 128
H = 128
C = 2

NC = 2
NS = 16
NW = NC * NS
EPW = E // NW
CH = 125
NCH = EPW // CH
NP = 10240
RPT = NP // NS
ZCH = 80

_mesh = plsc.VectorSubcoreMesh(core_axis_name="c", subcore_axis_name="s")


@functools.partial(
    pl.kernel,
    out_type=jax.ShapeDtypeStruct((NC, NP * 8), jnp.float32),
    mesh=_mesh,
    compiler_params=pltpu.CompilerParams(needs_layout_passes=False),
    scratch_types=[
        pltpu.VMEM_SHARED((NS * NP,), jnp.float32),
        pltpu.VMEM((EPW,), jnp.int32),
        pltpu.VMEM((NP,), jnp.float32),
        pltpu.VMEM((RPT,), jnp.float32),
        pltpu.VMEM((RPT,), jnp.float32),
        pltpu.VMEM((RPT * 8,), jnp.float32),
    ],
)
def _sc_degree(dst_hbm, out_hbm, stage, idx_v, hist_v, acc_v, tmp_v, obuf):
    c = lax.axis_index("c")
    s = lax.axis_index("s")
    w = c * NS + s
    one16 = jnp.full((16,), 1.0, jnp.float32)
    zer16 = jnp.zeros((16,), jnp.float32)

    def zh(i, carry):
        hist_v[pl.ds(i * 16, 16)] = zer16
        return carry

    lax.fori_loop(0, NP // 16, zh, 0)
    pltpu.sync_copy(dst_hbm.at[pl.ds(w * EPW, EPW)], idx_v)

    def step(j, carry):
        iv = idx_v[pl.ds(j * 16, 16)]
        plsc.addupdate_scatter(hist_v, [iv], one16)
        return carry

    lax.fori_loop(0, EPW // 16, step, 0)
    pltpu.sync_copy(hist_v, stage.at[pl.ds(s * NP, NP)])
    plsc.subcore_barrier()

    base = s * RPT

    def za(i, carry):
        acc_v[pl.ds(i * 16, 16)] = zer16
        return carry

    lax.fori_loop(0, RPT // 16, za, 0)

    def red(t, carry):
        pltpu.sync_copy(stage.at[pl.ds(t * NP + base, RPT)], tmp_v)

        def add(i, carry2):
            acc_v[pl.ds(i * 16, 16)] = (
                acc_v[pl.ds(i * 16, 16)] + tmp_v[pl.ds(i * 16, 16)]
            )
            return carry2

        lax.fori_loop(0, RPT // 16, add, 0)
        return carry

    lax.fori_loop(0, NS, red, 0)

    def put(k, carry):
        rows = (lax.iota(jnp.int32, 16) + k * 16) * 8
        plsc.store_scatter(obuf, [rows], acc_v[pl.ds(k * 16, 16)])
        return carry

    lax.fori_loop(0, RPT // 16, put, 0)
    pltpu.sync_copy(obuf, out_hbm.at[c, pl.ds(base * 8, RPT * 8)])


@functools.partial(
    pl.kernel,
    out_type=jax.ShapeDtypeStruct((NC, NP, H), jnp.float32),
    mesh=_mesh,
    scratch_types=[
        pltpu.VMEM_SHARED((NP, H), jnp.float32),
        pltpu.VMEM((CH,), jnp.int32),
        pltpu.VMEM((CH,), jnp.int32),
        pltpu.VMEM((CH,), jnp.int32),
        pltpu.VMEM((CH,), jnp.int32),
        pltpu.VMEM((CH, H), jnp.float32),
        pltpu.VMEM((CH, H), jnp.float32),
        pltpu.SemaphoreType.DMA,
        pltpu.SemaphoreType.DMA,
        pltpu.SemaphoreType.DMA,
        pltpu.SemaphoreType.DMA,
    ],
)
def _sc_scatter(g_hbm, src_hbm, dst_hbm, out_hbm, acc, sa0, sa1, da0, da1,
                buf0, buf1, sem0, sem1, semi0, semi1):
    c = lax.axis_index("c")
    s = lax.axis_index("s")
    w = c * NS + s
    zer16 = jnp.zeros((16,), jnp.float32)
    hb = H // 16

    def zb(i, carry):
        buf0[i // hb, pl.ds((i % hb) * 16, 16)] = zer16
        return carry

    lax.fori_loop(0, CH * hb, zb, 0)

    base = s * RPT

    def zc(k, carry):
        pltpu.sync_copy(buf0.at[pl.ds(0, ZCH)], acc.at[pl.ds(base + k * ZCH, ZCH)])
        return carry

    lax.fori_loop(0, RPT // ZCH, zc, 0)
    plsc.subcore_barrier()

    row0 = w * NCH

    pltpu.sync_copy(src_hbm.at[row0], sa0)
    pltpu.sync_copy(dst_hbm.at[row0], da0)
    pltpu.async_copy(g_hbm.at[sa0], buf0, sem0)
    pltpu.async_copy(src_hbm.at[row0 + 1], sa1, semi1)
    pltpu.async_copy(dst_hbm.at[row0 + 1], da1, semi1)
    nk2 = NCH // 2

    def step2(k, carry):
        j0 = row0 + 2 * k
        j1 = j0 + 1
        pltpu.make_async_copy(g_hbm.at[sa0], buf0, sem0).wait()
        pltpu.make_async_copy(src_hbm.at[j1], sa1, semi1).wait()
        pltpu.make_async_copy(dst_hbm.at[j1], da1, semi1).wait()
        pltpu.async_copy(g_hbm.at[sa1], buf1, sem1)
        pltpu.sync_copy(buf0, acc.at[da0], add=True)

        @pl.when(k < nk2 - 1)
        def _():
            pltpu.async_copy(src_hbm.at[j0 + 2], sa0, semi0)
            pltpu.async_copy(dst_hbm.at[j0 + 2], da0, semi0)

        pltpu.make_async_copy(g_hbm.at[sa1], buf1, sem1).wait()

        @pl.when(k < nk2 - 1)
        def _():
            pltpu.make_async_copy(src_hbm.at[j0 + 2], sa0, semi0).wait()
            pltpu.make_async_copy(dst_hbm.at[j0 + 2], da0, semi0).wait()
            pltpu.async_copy(g_hbm.at[sa0], buf0, sem0)

        pltpu.sync_copy(buf1, acc.at[da1], add=True)

        @pl.when(k < nk2 - 1)
        def _():
            pltpu.async_copy(src_hbm.at[j1 + 2], sa1, semi1)
            pltpu.async_copy(dst_hbm.at[j1 + 2], da1, semi1)

        return carry

    lax.fori_loop(0, nk2, step2, 0)
    plsc.subcore_barrier()
    pltpu.sync_copy(acc.at[pl.ds(base, RPT)], out_hbm.at[c].at[pl.ds(base, RPT)])


_R = 2000


def _tc1a_body(x_ref, w_ref, b_ref, h_ref):
    hv = lax.dot_general(
        x_ref[...], w_ref[...], (((1,), (1,)), ((), ())),
        preferred_element_type=jnp.float32)
    h_ref[...] = jnp.maximum(hv + b_ref[...], 0.0)


def _tc_stage1a(x, w_in, b_in):
    grid = (N // _R,)
    return pl.pallas_call(
        _tc1a_body,
        grid=grid,
        in_specs=[
            pl.BlockSpec((_R, D_IN), lambda i: (i, 0)),
            pl.BlockSpec((H, D_IN), lambda i: (0, 0)),
            pl.BlockSpec((1, H), lambda i: (0, 0)),
        ],
        out_specs=pl.BlockSpec((_R, H), lambda i: (i, 0)),
        out_shape=jax.ShapeDtypeStruct((N, H), jnp.float32),
    )(x, w_in, b_in)


def _tc1b_body(deg_ref, h_ref, g_ref, nrm_ref, inv_ref):
    d = deg_ref[0, :, :1] + deg_ref[1, :, :1]
    degc = jnp.maximum(d, 1.0)
    nrm = lax.rsqrt(degc)
    g_ref[...] = h_ref[...] * nrm
    nrm_ref[...] = nrm
    inv_ref[...] = jnp.sqrt(degc)


def _tc_stage1b(deg_parts, h):
    grid = (N // _R,)
    return pl.pallas_call(
        _tc1b_body,
        grid=grid,
        in_specs=[
            pl.BlockSpec((NC, _R, 8), lambda i: (0, i, 0)),
            pl.BlockSpec((_R, H), lambda i: (i, 0)),
        ],
        out_specs=[
            pl.BlockSpec((_R, H), lambda i: (i, 0)),
            pl.BlockSpec((_R, 1), lambda i: (i, 0)),
            pl.BlockSpec((_R, 1), lambda i: (i, 0)),
        ],
        out_shape=[
            jax.ShapeDtypeStruct((N, H), jnp.float32),
            jax.ShapeDtypeStruct((N, 1), jnp.float32),
            jax.ShapeDtypeStruct((N, 1), jnp.float32),
        ],
    )(deg_parts, h)


def _tc2_body(g_ref, sp_ref, nrm_ref, inv_ref, w1_ref, w2_ref, bc_ref,
              wo_ref, bo_ref, out_ref):
    sm = sp_ref[0] + sp_ref[1]
    dn = (((1,), (1,)), ((), ()))
    p = lax.dot_general(g_ref[...], w1_ref[...], dn,
                        preferred_element_type=jnp.float32)
    q = lax.dot_general(sm, w2_ref[...], dn,
                        preferred_element_type=jnp.float32)
    h2 = jnp.maximum(p * inv_ref[...] - q * nrm_ref[...] + bc_ref[...], 0.0)
    out_ref[...] = (
        lax.dot_general(h2, wo_ref[...], dn,
                        preferred_element_type=jnp.float32)
        + bo_ref[...]
    )


def _tc_stage2(g, s_parts, nrm, inv, w1_t, w2_t, b_cheb, w_out_t, b_out):
    grid = (N // _R,)
    return pl.pallas_call(
        _tc2_body,
        grid=grid,
        in_specs=[
            pl.BlockSpec((_R, H), lambda i: (i, 0)),
            pl.BlockSpec((NC, _R, H), lambda i: (0, i, 0)),
            pl.BlockSpec((_R, 1), lambda i: (i, 0)),
            pl.BlockSpec((_R, 1), lambda i: (i, 0)),
            pl.BlockSpec((H, H), lambda i: (0, 0)),
            pl.BlockSpec((H, H), lambda i: (0, 0)),
            pl.BlockSpec((1, H), lambda i: (0, 0)),
            pl.BlockSpec((C, H), lambda i: (0, 0)),
            pl.BlockSpec((1, C), lambda i: (0, 0)),
        ],
        out_specs=pl.BlockSpec((_R, C), lambda i: (i, 0)),
        out_shape=jax.ShapeDtypeStruct((N, C), jnp.float32),
    )(g, s_parts, nrm, inv, w1_t, w2_t, b_cheb, w_out_t, b_out)


def kernel(x, edge_index, W_in, b_in, W_cheb, b_cheb, W_out, b_out):
    src2 = edge_index[0].reshape(E // CH, CH)
    dst2 = edge_index[1].reshape(E // CH, CH)

    deg_parts = _sc_degree(edge_index[1]).reshape(NC, NP, 8)
    h = _tc_stage1a(x, W_in, b_in.reshape(1, H))
    g, nrm, inv = _tc_stage1b(deg_parts, h)
    s_parts = _sc_scatter(g, src2, dst2)
    out = _tc_stage2(
        g, s_parts, nrm, inv,
        W_cheb[:, :H], W_cheb[:, H:], b_cheb.reshape(1, H),
        W_out, b_out.reshape(1, C),
    )
    return out


# speedup vs baseline: 1.0031x; 1.0031x over previous
import functools

import jax
import jax.numpy as jnp
from jax import lax
from jax.experimental import pallas as pl
from jax.experimental.pallas import tpu as pltpu
from jax.experimental.pallas import tpu_sc as plsc

N = 10000
E = 320000
D_IN = 128
H = 128
C = 2

NC = 2
NS = 16
NW = NC * NS
EPW = E // NW
CH = 125
NCH = EPW // CH
NP = 10240
RPT = NP // NS
ZCH = 80

_mesh = plsc.VectorSubcoreMesh(core_axis_name="c", subcore_axis_name="s")


@functools.partial(
    pl.kernel,
    out_type=jax.ShapeDtypeStruct((NC, NP * 8), jnp.float32),
    mesh=_mesh,
    compiler_params=pltpu.CompilerParams(needs_layout_passes=False),
    scratch_types=[
        pltpu.VMEM_SHARED((NS * NP,), jnp.float32),
        pltpu.VMEM((EPW,), jnp.int32),
        pltpu.VMEM((NP,), jnp.float32),
        pltpu.VMEM((RPT,), jnp.float32),
        pltpu.VMEM((RPT,), jnp.float32),
        pltpu.VMEM((RPT * 8,), jnp.float32),
    ],
)
def _sc_degree(dst_hbm, out_hbm, stage, idx_v, hist_v, acc_v, tmp_v, obuf):
    c = lax.axis_index("c")
    s = lax.axis_index("s")
    w = c * NS + s
    one16 = jnp.full((16,), 1.0, jnp.float32)
    zer16 = jnp.zeros((16,), jnp.float32)

    def zh(i, carry):
        hist_v[pl.ds(i * 16, 16)] = zer16
        return carry

    lax.fori_loop(0, NP // 16, zh, 0)
    pltpu.sync_copy(dst_hbm.at[pl.ds(w * EPW, EPW)], idx_v)

    def step(j, carry):
        iv = idx_v[pl.ds(j * 16, 16)]
        plsc.addupdate_scatter(hist_v, [iv], one16)
        return carry

    lax.fori_loop(0, EPW // 16, step, 0)
    pltpu.sync_copy(hist_v, stage.at[pl.ds(s * NP, NP)])
    plsc.subcore_barrier()

    base = s * RPT

    def za(i, carry):
        acc_v[pl.ds(i * 16, 16)] = zer16
        return carry

    lax.fori_loop(0, RPT // 16, za, 0)

    def red(t, carry):
        pltpu.sync_copy(stage.at[pl.ds(t * NP + base, RPT)], tmp_v)

        def add(i, carry2):
            acc_v[pl.ds(i * 16, 16)] = (
                acc_v[pl.ds(i * 16, 16)] + tmp_v[pl.ds(i * 16, 16)]
            )
            return carry2

        lax.fori_loop(0, RPT // 16, add, 0)
        return carry

    lax.fori_loop(0, NS, red, 0)

    def put(k, carry):
        rows = (lax.iota(jnp.int32, 16) + k * 16) * 8
        plsc.store_scatter(obuf, [rows], acc_v[pl.ds(k * 16, 16)])
        return carry

    lax.fori_loop(0, RPT // 16, put, 0)
    pltpu.sync_copy(obuf, out_hbm.at[c, pl.ds(base * 8, RPT * 8)])


@functools.partial(
    pl.kernel,
    out_type=jax.ShapeDtypeStruct((NC, NP, H), jnp.float32),
    mesh=_mesh,
    scratch_types=[
        pltpu.VMEM_SHARED((NP, H), jnp.float32),
        [pltpu.VMEM((CH,), jnp.int32) for _ in range(4)],
        [pltpu.VMEM((CH,), jnp.int32) for _ in range(4)],
        [pltpu.VMEM((CH, H), jnp.float32) for _ in range(2)],
        [pltpu.SemaphoreType.DMA for _ in range(2)],
        [pltpu.SemaphoreType.DMA for _ in range(2)],
        [pltpu.SemaphoreType.DMA for _ in range(4)],
    ],
)
def _sc_scatter(g_hbm, src_hbm, dst_hbm, out_hbm, acc, sa, da,
                buf, semg, sems, semi):
    c = lax.axis_index("c")
    s = lax.axis_index("s")
    w = c * NS + s
    zer16 = jnp.zeros((16,), jnp.float32)
    hb = H // 16

    def zb(i, carry):
        buf[0][i // hb, pl.ds((i % hb) * 16, 16)] = zer16
        return carry

    lax.fori_loop(0, CH * hb, zb, 0)

    base = s * RPT

    def zc(k, carry):
        pltpu.sync_copy(buf[0].at[pl.ds(0, ZCH)], acc.at[pl.ds(base + k * ZCH, ZCH)])
        return carry

    lax.fori_loop(0, RPT // ZCH, zc, 0)
    plsc.subcore_barrier()

    row0 = w * NCH

    for q in range(3):
        pltpu.async_copy(src_hbm.at[row0 + q], sa[q], semi[q])
        pltpu.async_copy(dst_hbm.at[row0 + q], da[q], semi[q])
    pltpu.make_async_copy(src_hbm.at[row0], sa[0], semi[0]).wait()
    pltpu.make_async_copy(dst_hbm.at[row0], da[0], semi[0]).wait()
    pltpu.async_copy(g_hbm.at[sa[0]], buf[0], semg[0])

    def step4(k, carry):
        for pos in range(4):
            p = pos % 2
            jj = 4 * k + pos
            j = row0 + jj
            pltpu.make_async_copy(g_hbm.at[sa[pos]], buf[p], semg[p]).wait()
            pltpu.async_copy(buf[p], acc.at[da[pos]], sems[p], add=True)

            def wait_prev():
                qm = (pos - 1) % 4
                pltpu.make_async_copy(
                    buf[1 - p], acc.at[da[qm]], sems[1 - p]).wait()

            if pos == 0:
                @pl.when(k > 0)
                def _():
                    wait_prev()
            else:
                wait_prev()

            qn = (pos + 1) % 4

            @pl.when(jj + 1 < NCH)
            def _():
                pltpu.make_async_copy(src_hbm.at[j + 1], sa[qn], semi[qn]).wait()
                pltpu.make_async_copy(dst_hbm.at[j + 1], da[qn], semi[qn]).wait()
                pltpu.async_copy(g_hbm.at[sa[qn]], buf[1 - p], semg[1 - p])

            qp = (pos + 3) % 4

            @pl.when(jj + 3 < NCH)
            def _():
                pltpu.async_copy(src_hbm.at[j + 3], sa[qp], semi[qp])
                pltpu.async_copy(dst_hbm.at[j + 3], da[qp], semi[qp])

        return carry

    lax.fori_loop(0, NCH // 4, step4, 0)
    pltpu.make_async_copy(buf[1], acc.at[da[3]], sems[1]).wait()
    plsc.subcore_barrier()
    pltpu.sync_copy(acc.at[pl.ds(base, RPT)], out_hbm.at[c].at[pl.ds(base, RPT)])


_R = 2000


def _tc1a_body(x_ref, w_ref, b_ref, h_ref):
    hv = lax.dot_general(
        x_ref[...], w_ref[...], (((1,), (1,)), ((), ())),
        preferred_element_type=jnp.float32)
    h_ref[...] = jnp.maximum(hv + b_ref[...], 0.0)


def _tc_stage1a(x, w_in, b_in):
    grid = (N // _R,)
    return pl.pallas_call(
        _tc1a_body,
        grid=grid,
        in_specs=[
            pl.BlockSpec((_R, D_IN), lambda i: (i, 0)),
            pl.BlockSpec((H, D_IN), lambda i: (0, 0)),
            pl.BlockSpec((1, H), lambda i: (0, 0)),
        ],
        out_specs=pl.BlockSpec((_R, H), lambda i: (i, 0)),
        out_shape=jax.ShapeDtypeStruct((N, H), jnp.float32),
    )(x, w_in, b_in)


def _tc1b_body(deg_ref, h_ref, g_ref, nrm_ref, inv_ref):
    d = deg_ref[0, :, :1] + deg_ref[1, :, :1]
    degc = jnp.maximum(d, 1.0)
    nrm = lax.rsqrt(degc)
    g_ref[...] = h_ref[...] * nrm
    nrm_ref[...] = nrm
    inv_ref[...] = jnp.sqrt(degc)


def _tc_stage1b(deg_parts, h):
    grid = (N // _R,)
    return pl.pallas_call(
        _tc1b_body,
        grid=grid,
        in_specs=[
            pl.BlockSpec((NC, _R, 8), lambda i: (0, i, 0)),
            pl.BlockSpec((_R, H), lambda i: (i, 0)),
        ],
        out_specs=[
            pl.BlockSpec((_R, H), lambda i: (i, 0)),
            pl.BlockSpec((_R, 1), lambda i: (i, 0)),
            pl.BlockSpec((_R, 1), lambda i: (i, 0)),
        ],
        out_shape=[
            jax.ShapeDtypeStruct((N, H), jnp.float32),
            jax.ShapeDtypeStruct((N, 1), jnp.float32),
            jax.ShapeDtypeStruct((N, 1), jnp.float32),
        ],
    )(deg_parts, h)


def _tc2_body(g_ref, sp_ref, nrm_ref, inv_ref, w1_ref, w2_ref, bc_ref,
              wo_ref, bo_ref, out_ref):
    sm = sp_ref[0] + sp_ref[1]
    dn = (((1,), (1,)), ((), ()))
    p = lax.dot_general(g_ref[...], w1_ref[...], dn,
                        preferred_element_type=jnp.float32)
    q = lax.dot_general(sm, w2_ref[...], dn,
                        preferred_element_type=jnp.float32)
    h2 = jnp.maximum(p * inv_ref[...] - q * nrm_ref[...] + bc_ref[...], 0.0)
    out_ref[...] = (
        lax.dot_general(h2, wo_ref[...], dn,
                        preferred_element_type=jnp.float32)
        + bo_ref[...]
    )


def _tc_stage2(g, s_parts, nrm, inv, w1_t, w2_t, b_cheb, w_out_t, b_out):
    grid = (N // _R,)
    return pl.pallas_call(
        _tc2_body,
        grid=grid,
        in_specs=[
            pl.BlockSpec((_R, H), lambda i: (i, 0)),
            pl.BlockSpec((NC, _R, H), lambda i: (0, i, 0)),
            pl.BlockSpec((_R, 1), lambda i: (i, 0)),
            pl.BlockSpec((_R, 1), lambda i: (i, 0)),
            pl.BlockSpec((H, H), lambda i: (0, 0)),
            pl.BlockSpec((H, H), lambda i: (0, 0)),
            pl.BlockSpec((1, H), lambda i: (0, 0)),
            pl.BlockSpec((C, H), lambda i: (0, 0)),
            pl.BlockSpec((1, C), lambda i: (0, 0)),
        ],
        out_specs=pl.BlockSpec((_R, C), lambda i: (i, 0)),
        out_shape=jax.ShapeDtypeStruct((N, C), jnp.float32),
    )(g, s_parts, nrm, inv, w1_t, w2_t, b_cheb, w_out_t, b_out)


def kernel(x, edge_index, W_in, b_in, W_cheb, b_cheb, W_out, b_out):
    src2 = edge_index[0].reshape(E // CH, CH)
    dst2 = edge_index[1].reshape(E // CH, CH)

    deg_parts = _sc_degree(edge_index[1]).reshape(NC, NP, 8)
    h = _tc_stage1a(x, W_in, b_in.reshape(1, H))
    g, nrm, inv = _tc_stage1b(deg_parts, h)
    s_parts = _sc_scatter(g, src2, dst2)
    out = _tc_stage2(
        g, s_parts, nrm, inv,
        W_cheb[:, :H], W_cheb[:, H:], b_cheb.reshape(1, H),
        W_out, b_out.reshape(1, C),
    )
    return out


# speedup vs baseline: 1.0054x; 1.0023x over previous
import functools

import jax
import jax.numpy as jnp
from jax import lax
from jax.experimental import pallas as pl
from jax.experimental.pallas import tpu as pltpu
from jax.experimental.pallas import tpu_sc as plsc

N = 10000
E = 320000
D_IN = 128
H = 128
C = 2

NC = 2
NS = 16
NW = NC * NS
EPW = E // NW
CH = 125
NCH = EPW // CH
NP = 10240
RPT = NP // NS
ZCH = 80

_mesh = plsc.VectorSubcoreMesh(core_axis_name="c", subcore_axis_name="s")


@functools.partial(
    pl.kernel,
    out_type=jax.ShapeDtypeStruct((NC, NP * 8), jnp.float32),
    mesh=_mesh,
    compiler_params=pltpu.CompilerParams(needs_layout_passes=False),
    scratch_types=[
        pltpu.VMEM_SHARED((NS * NP,), jnp.float32),
        pltpu.VMEM((EPW,), jnp.int32),
        pltpu.VMEM((NP,), jnp.float32),
        pltpu.VMEM((RPT,), jnp.float32),
        pltpu.VMEM((RPT,), jnp.float32),
        pltpu.VMEM((RPT * 8,), jnp.float32),
    ],
)
def _sc_degree(dst_hbm, out_hbm, stage, idx_v, hist_v, acc_v, tmp_v, obuf):
    c = lax.axis_index("c")
    s = lax.axis_index("s")
    w = c * NS + s
    one16 = jnp.full((16,), 1.0, jnp.float32)
    zer16 = jnp.zeros((16,), jnp.float32)

    def zh(i, carry):
        hist_v[pl.ds(i * 16, 16)] = zer16
        return carry

    lax.fori_loop(0, NP // 16, zh, 0)
    pltpu.sync_copy(dst_hbm.at[pl.ds(w * EPW, EPW)], idx_v)

    def step(j, carry):
        iv = idx_v[pl.ds(j * 16, 16)]
        plsc.addupdate_scatter(hist_v, [iv], one16)
        return carry

    lax.fori_loop(0, EPW // 16, step, 0)
    pltpu.sync_copy(hist_v, stage.at[pl.ds(s * NP, NP)])
    plsc.subcore_barrier()

    base = s * RPT

    def za(i, carry):
        acc_v[pl.ds(i * 16, 16)] = zer16
        return carry

    lax.fori_loop(0, RPT // 16, za, 0)

    def red(t, carry):
        pltpu.sync_copy(stage.at[pl.ds(t * NP + base, RPT)], tmp_v)

        def add(i, carry2):
            acc_v[pl.ds(i * 16, 16)] = (
                acc_v[pl.ds(i * 16, 16)] + tmp_v[pl.ds(i * 16, 16)]
            )
            return carry2

        lax.fori_loop(0, RPT // 16, add, 0)
        return carry

    lax.fori_loop(0, NS, red, 0)

    def put(k, carry):
        rows = (lax.iota(jnp.int32, 16) + k * 16) * 8
        plsc.store_scatter(obuf, [rows], acc_v[pl.ds(k * 16, 16)])
        return carry

    lax.fori_loop(0, RPT // 16, put, 0)
    pltpu.sync_copy(obuf, out_hbm.at[c, pl.ds(base * 8, RPT * 8)])


@functools.partial(
    pl.kernel,
    out_type=jax.ShapeDtypeStruct((NC, NP, H), jnp.float32),
    mesh=_mesh,
    scratch_types=[
        pltpu.VMEM_SHARED((NP, H), jnp.float32),
        pltpu.VMEM((NCH, CH), jnp.int32),
        [pltpu.VMEM((CH,), jnp.int32) for _ in range(4)],
        [pltpu.VMEM((CH, H), jnp.float32) for _ in range(2)],
        [pltpu.SemaphoreType.DMA for _ in range(2)],
        [pltpu.SemaphoreType.DMA for _ in range(2)],
        [pltpu.SemaphoreType.DMA for _ in range(4)],
    ],
)
def _sc_scatter(g_hbm, src_hbm, dst_hbm, out_hbm, acc, sidx, da,
                buf, semg, sems, semi):
    c = lax.axis_index("c")
    s = lax.axis_index("s")
    w = c * NS + s
    zer16 = jnp.zeros((16,), jnp.float32)
    hb = H // 16

    def zb(i, carry):
        buf[0][i // hb, pl.ds((i % hb) * 16, 16)] = zer16
        return carry

    lax.fori_loop(0, CH * hb, zb, 0)

    base = s * RPT

    def zc(k, carry):
        pltpu.sync_copy(buf[0].at[pl.ds(0, ZCH)], acc.at[pl.ds(base + k * ZCH, ZCH)])
        return carry

    lax.fori_loop(0, RPT // ZCH, zc, 0)
    plsc.subcore_barrier()

    row0 = w * NCH

    pltpu.sync_copy(src_hbm.at[pl.ds(row0, NCH)], sidx)
    for q in range(3):
        pltpu.async_copy(dst_hbm.at[row0 + q], da[q], semi[q])
    pltpu.async_copy(g_hbm.at[sidx.at[0]], buf[0], semg[0])
    pltpu.make_async_copy(dst_hbm.at[row0], da[0], semi[0]).wait()

    def step4(k, carry):
        for pos in range(4):
            p = pos % 2
            jj = 4 * k + pos
            j = row0 + jj
            pltpu.make_async_copy(g_hbm.at[sidx.at[jj]], buf[p], semg[p]).wait()
            pltpu.async_copy(buf[p], acc.at[da[pos]], sems[p], add=True)

            def wait_prev():
                qm = (pos - 1) % 4
                pltpu.make_async_copy(
                    buf[1 - p], acc.at[da[qm]], sems[1 - p]).wait()

            if pos == 0:
                @pl.when(k > 0)
                def _():
                    wait_prev()
            else:
                wait_prev()

            qn = (pos + 1) % 4

            @pl.when(jj + 1 < NCH)
            def _():
                pltpu.make_async_copy(dst_hbm.at[j + 1], da[qn], semi[qn]).wait()
                pltpu.async_copy(g_hbm.at[sidx.at[jj + 1]], buf[1 - p], semg[1 - p])

            qp = (pos + 3) % 4

            @pl.when(jj + 3 < NCH)
            def _():
                pltpu.async_copy(dst_hbm.at[j + 3], da[qp], semi[qp])

        return carry

    lax.fori_loop(0, NCH // 4, step4, 0)
    pltpu.make_async_copy(buf[1], acc.at[da[3]], sems[1]).wait()
    plsc.subcore_barrier()
    pltpu.sync_copy(acc.at[pl.ds(base, RPT)], out_hbm.at[c].at[pl.ds(base, RPT)])


_R = 2000


def _tc1a_body(x_ref, w_ref, b_ref, h_ref):
    hv = lax.dot_general(
        x_ref[...], w_ref[...], (((1,), (1,)), ((), ())),
        preferred_element_type=jnp.float32)
    h_ref[...] = jnp.maximum(hv + b_ref[...], 0.0)


def _tc_stage1a(x, w_in, b_in):
    grid = (N // _R,)
    return pl.pallas_call(
        _tc1a_body,
        grid=grid,
        in_specs=[
            pl.BlockSpec((_R, D_IN), lambda i: (i, 0)),
            pl.BlockSpec((H, D_IN), lambda i: (0, 0)),
            pl.BlockSpec((1, H), lambda i: (0, 0)),
        ],
        out_specs=pl.BlockSpec((_R, H), lambda i: (i, 0)),
        out_shape=jax.ShapeDtypeStruct((N, H), jnp.float32),
    )(x, w_in, b_in)


def _tc1b_body(deg_ref, h_ref, g_ref, nrm_ref, inv_ref):
    d = deg_ref[0, :, :1] + deg_ref[1, :, :1]
    degc = jnp.maximum(d, 1.0)
    nrm = lax.rsqrt(degc)
    g_ref[...] = h_ref[...] * nrm
    nrm_ref[...] = nrm
    inv_ref[...] = jnp.sqrt(degc)


def _tc_stage1b(deg_parts, h):
    grid = (N // _R,)
    return pl.pallas_call(
        _tc1b_body,
        grid=grid,
        in_specs=[
            pl.BlockSpec((NC, _R, 8), lambda i: (0, i, 0)),
            pl.BlockSpec((_R, H), lambda i: (i, 0)),
        ],
        out_specs=[
            pl.BlockSpec((_R, H), lambda i: (i, 0)),
            pl.BlockSpec((_R, 1), lambda i: (i, 0)),
            pl.BlockSpec((_R, 1), lambda i: (i, 0)),
        ],
        out_shape=[
            jax.ShapeDtypeStruct((N, H), jnp.float32),
            jax.ShapeDtypeStruct((N, 1), jnp.float32),
            jax.ShapeDtypeStruct((N, 1), jnp.float32),
        ],
    )(deg_parts, h)


def _tc2_body(g_ref, sp_ref, nrm_ref, inv_ref, w1_ref, w2_ref, bc_ref,
              wo_ref, bo_ref, out_ref):
    sm = sp_ref[0] + sp_ref[1]
    dn = (((1,), (1,)), ((), ()))
    p = lax.dot_general(g_ref[...], w1_ref[...], dn,
                        preferred_element_type=jnp.float32)
    q = lax.dot_general(sm, w2_ref[...], dn,
                        preferred_element_type=jnp.float32)
    h2 = jnp.maximum(p * inv_ref[...] - q * nrm_ref[...] + bc_ref[...], 0.0)
    out_ref[...] = (
        lax.dot_general(h2, wo_ref[...], dn,
                        preferred_element_type=jnp.float32)
        + bo_ref[...]
    )


def _tc_stage2(g, s_parts, nrm, inv, w1_t, w2_t, b_cheb, w_out_t, b_out):
    grid = (N // _R,)
    return pl.pallas_call(
        _tc2_body,
        grid=grid,
        in_specs=[
            pl.BlockSpec((_R, H), lambda i: (i, 0)),
            pl.BlockSpec((NC, _R, H), lambda i: (0, i, 0)),
            pl.BlockSpec((_R, 1), lambda i: (i, 0)),
            pl.BlockSpec((_R, 1), lambda i: (i, 0)),
            pl.BlockSpec((H, H), lambda i: (0, 0)),
            pl.BlockSpec((H, H), lambda i: (0, 0)),
            pl.BlockSpec((1, H), lambda i: (0, 0)),
            pl.BlockSpec((C, H), lambda i: (0, 0)),
            pl.BlockSpec((1, C), lambda i: (0, 0)),
        ],
        out_specs=pl.BlockSpec((_R, C), lambda i: (i, 0)),
        out_shape=jax.ShapeDtypeStruct((N, C), jnp.float32),
    )(g, s_parts, nrm, inv, w1_t, w2_t, b_cheb, w_out_t, b_out)


def kernel(x, edge_index, W_in, b_in, W_cheb, b_cheb, W_out, b_out):
    src2 = edge_index[0].reshape(E // CH, CH)
    dst2 = edge_index[1].reshape(E // CH, CH)

    deg_parts = _sc_degree(edge_index[1]).reshape(NC, NP, 8)
    h = _tc_stage1a(x, W_in, b_in.reshape(1, H))
    g, nrm, inv = _tc_stage1b(deg_parts, h)
    s_parts = _sc_scatter(g, src2, dst2)
    out = _tc_stage2(
        g, s_parts, nrm, inv,
        W_cheb[:, :H], W_cheb[:, H:], b_cheb.reshape(1, H),
        W_out, b_out.reshape(1, C),
    )
    return out
